# baseline (speedup 1.0000x reference)
import functools

import jax
import jax.numpy as jnp
from jax import lax
from jax.experimental import pallas as pl
from jax.experimental.pallas import tpu as pltpu
import jax.experimental.pallas.tpu_sc as plsc

N = 10000
D = 128
H1 = 256
E = 320000

NC = 2
NS = 16
NW = NC * NS
K = 128
CH0 = 160
CH1 = 0
CHG = 32
TOT_CH = NS * (CH0 + CH1)
EPAD = TOT_CH * K
NPAD = 10112
RPS = NPAD // NS

CHD = TOT_CH // NW

_ZCHUNKS = [(0, 128), (128, 128), (256, 128), (384, 128), (512, 120)]


def _deg_body(dst_hbm, ones_hbm, zeros_hbm, out_hbm, dst_v, ones_v, row_v,
              acc_d):
    c = lax.axis_index("c")
    s = lax.axis_index("s")
    wid = s * NC + c
    pltpu.sync_copy(dst_hbm.at[pl.ds(wid * CHD, CHD)], dst_v)
    pltpu.sync_copy(ones_hbm, ones_v)
    pltpu.sync_copy(zeros_hbm, row_v)
    pltpu.sync_copy(row_v, acc_d.at[pl.ds(s * RPS, RPS)])
    plsc.subcore_barrier()

    def body(j, carry):
        pltpu.sync_copy(ones_v, acc_d.at[dst_v.at[j]], add=True)
        return carry

    lax.fori_loop(0, CHD, body, 0)
    plsc.subcore_barrier()
    pltpu.sync_copy(acc_d.at[pl.ds(s * RPS, RPS)], row_v)
    pltpu.sync_copy(row_v, out_hbm.at[pl.ds(c * NPAD + s * RPS, RPS)])


@functools.cache
def _deg_kernel():
    mesh = plsc.VectorSubcoreMesh(
        core_axis_name="c", subcore_axis_name="s",
        num_cores=NC, num_subcores=NS)
    return pl.kernel(
        _deg_body,
        out_type=jax.ShapeDtypeStruct((NC * NPAD,), jnp.float32),
        mesh=mesh,
        scratch_types=[
            pltpu.VMEM((CHD, K), jnp.int32),
            pltpu.VMEM((K,), jnp.float32),
            pltpu.VMEM((RPS,), jnp.float32),
            pltpu.VMEM_SHARED((NPAD,), jnp.float32),
        ],
    )


def _agg_body(g_hbm, src_hbm, dst_hbm, zblk_hbm, out_hbm,
              src_v, db, rb, acc, sg0, sg1, sd0, sd1):
    c = lax.axis_index("c")
    s = lax.axis_index("s")
    pltpu.sync_copy(zblk_hbm, rb.at[0])
    for off, sz in _ZCHUNKS:
        pltpu.sync_copy(rb.at[0, pl.ds(0, sz)],
                        acc.at[pl.ds(s * RPS + off, sz)])
    plsc.subcore_barrier()

    sg = (sg0, sg1)
    sd = (sd0, sd1)

    def run_group(base, n):
        pltpu.sync_copy(src_hbm.at[pl.ds(base, n)], src_v.at[pl.ds(0, n)])

        def issue(k, b):
            pltpu.async_copy(g_hbm.at[src_v.at[k]], rb.at[b], sg[b])
            pltpu.async_copy(dst_hbm.at[base + k], db.at[b], sd[b])

        issue(0, 0)

        def body(i, carry):
            for b in range(2):
                k = 2 * i + b

                @pl.when(k + 1 < n)
                def _():
                    issue(k + 1, 1 - b)

                pltpu.make_async_copy(g_hbm.at[src_v.at[k]], rb.at[b],
                                      sg[b]).wait()
                pltpu.make_async_copy(dst_hbm.at[base + k], db.at[b],
                                      sd[b]).wait()
                pltpu.sync_copy(rb.at[b], acc.at[db.at[b]], add=True)
            return carry

        lax.fori_loop(0, n // 2, body, 0)

    @pl.when(c == 0)
    def _():
        for off in range(0, CH0, CHG):
            run_group(s * (CH0 + CH1) + off, CHG)

    if CH1:
        @pl.when(c == 1)
        def _():
            run_group(s * (CH0 + CH1) + CH0, CH1)

    plsc.subcore_barrier()
    for off, sz in _ZCHUNKS:
        pltpu.sync_copy(acc.at[pl.ds(s * RPS + off, sz)],
                        rb.at[0, pl.ds(0, sz)])
        pltpu.sync_copy(rb.at[0, pl.ds(0, sz)],
                        out_hbm.at[c, pl.ds(s * RPS + off, sz)])


@functools.cache
def _agg_kernel():
    mesh = plsc.VectorSubcoreMesh(
        core_axis_name="c", subcore_axis_name="s",
        num_cores=NC, num_subcores=NS)
    return pl.kernel(
        _agg_body,
        out_type=jax.ShapeDtypeStruct((NC, NPAD, D), jnp.float32),
        mesh=mesh,
        scratch_types=[
            pltpu.VMEM((CHG, K), jnp.int32),
            pltpu.VMEM((2, K), jnp.int32),
            pltpu.VMEM((2, K, D), jnp.float32),
            pltpu.VMEM_SHARED((NPAD, D), jnp.float32),
        ] + [pltpu.SemaphoreType.DMA] * 4,
    )


_GRID = 8
_R = NPAD // _GRID

_DOT = functools.partial(
    lax.dot_general,
    dimension_numbers=(((1,), (0,)), ((), ())),
    precision=lax.Precision.HIGHEST,
    preferred_element_type=jnp.float32,
)


def _tc1_body(z_ref, d0_ref, d1_ref, g1_ref, dinv_ref):
    deg = d0_ref[...] + d1_ref[...] + 1.0
    dinv = lax.rsqrt(deg)
    dinv_ref[...] = dinv
    g1_ref[...] = dinv * z_ref[...]


def _tc2_body(s_ref, g1_ref, dinv_ref, w1_ref, b1_ref, w2_ref, g2_ref):
    dinv = dinv_ref[...]
    m = dinv * (s_ref[0] + s_ref[1] + g1_ref[...])
    h = jnp.maximum(_DOT(m, w1_ref[...]) + b1_ref[...], 0.0)
    g2_ref[...] = dinv * _DOT(h, w2_ref[...])


def _tc3_body(s_ref, g2_ref, dinv_ref, b2_ref, w3_ref, g3_ref):
    dinv = dinv_ref[...]
    h = jnp.maximum(dinv * (s_ref[0] + s_ref[1] + g2_ref[...]) + b2_ref[...],
                    0.0)
    g3_ref[...] = dinv * _DOT(h, w3_ref[...])


def _tc4_body(s_ref, g3_ref, dinv_ref, b3_ref, out_ref):
    out_ref[...] = (dinv_ref[...] * (s_ref[0] + s_ref[1] + g3_ref[...])
                    + b3_ref[...])


def _rows(bs):
    return pl.BlockSpec(bs, lambda i: (i,) + (0,) * (len(bs) - 1))


def _full(shape):
    return pl.BlockSpec(shape, lambda i: (0,) * len(shape))


def _srows():
    return pl.BlockSpec((NC, _R, D), lambda i: (0, i, 0))


_tc1 = pl.pallas_call(
    _tc1_body,
    grid=(_GRID,),
    in_specs=[_rows((_R, D)), _rows((_R, 1)), _rows((_R, 1))],
    out_specs=[_rows((_R, D)), _rows((_R, 1))],
    out_shape=[jax.ShapeDtypeStruct((NPAD, D), jnp.float32),
               jax.ShapeDtypeStruct((NPAD, 1), jnp.float32)],
)

_tc2 = pl.pallas_call(
    _tc2_body,
    grid=(_GRID,),
    in_specs=[_srows(), _rows((_R, D)), _rows((_R, 1)),
              _full((D, H1)), _full((1, H1)), _full((H1, D))],
    out_specs=_rows((_R, D)),
    out_shape=jax.ShapeDtypeStruct((NPAD, D), jnp.float32),
)

_tc3 = pl.pallas_call(
    _tc3_body,
    grid=(_GRID,),
    in_specs=[_srows(), _rows((_R, D)), _rows((_R, 1)),
              _full((1, D)), _full((D, D))],
    out_specs=_rows((_R, D)),
    out_shape=jax.ShapeDtypeStruct((NPAD, D), jnp.float32),
)

_tc4 = pl.pallas_call(
    _tc4_body,
    grid=(_GRID,),
    in_specs=[_srows(), _rows((_R, D)), _rows((_R, 1)), _full((1, D))],
    out_specs=_rows((_R, D)),
    out_shape=jax.ShapeDtypeStruct((NPAD, D), jnp.float32),
)


def kernel(z, edge_index, W1, b1, W2, b2, W3, b3):
    z_pad = jnp.pad(z, ((0, NPAD - N), (0, 0)))
    pad_idx = jnp.full((EPAD - E,), N, dtype=jnp.int32)
    srcp = jnp.concatenate([edge_index[0], pad_idx]).reshape(TOT_CH, K)
    dstp = jnp.concatenate([edge_index[1], pad_idx]).reshape(TOT_CH, K)
    ones_k = jnp.ones((K,), jnp.float32)
    zeros_n = jnp.zeros((RPS,), jnp.float32)
    zblk = jnp.zeros((K, D), jnp.float32)

    degp = _deg_kernel()(dstp, ones_k, zeros_n)
    d0 = degp[:NPAD][:, None]
    d1 = degp[NPAD:][:, None]

    agg = _agg_kernel()
    g1, dinv = _tc1(z_pad, d0, d1)
    s1 = agg(g1, srcp, dstp, zblk)
    g2 = _tc2(s1, g1, dinv, W1, b1[None, :], W2)
    s2 = agg(g2, srcp, dstp, zblk)
    g3 = _tc3(s2, g2, dinv, b2[None, :], W3)
    s3 = agg(g3, srcp, dstp, zblk)
    out = _tc4(s3, g3, dinv, b3[None, :])
    return out[:N]


# speedup vs baseline: 1.2214x; 1.2214x over previous
import functools

import jax
import jax.numpy as jnp
from jax import lax
from jax.experimental import pallas as pl
from jax.experimental.pallas import tpu as pltpu
import jax.experimental.pallas.tpu_sc as plsc

N = 10000
D = 128
H1 = 256
E = 320000

NC = 2
NS = 16
NW = NC * NS
K = 128
CH0 = 120
CH1 = 40
CHG = 40
TOT_CH = NS * (CH0 + CH1)
EPAD = TOT_CH * K
NPAD = 10112
RPS = NPAD // NS

CHD = TOT_CH // NW

_ZCHUNKS = [(0, 128), (128, 128), (256, 128), (384, 128), (512, 120)]


def _deg_body(dst_hbm, ones_hbm, zeros_hbm, out_hbm, dst_v, ones_v, row_v,
              acc_d):
    c = lax.axis_index("c")
    s = lax.axis_index("s")
    wid = s * NC + c
    pltpu.sync_copy(dst_hbm.at[pl.ds(wid * CHD, CHD)], dst_v)
    pltpu.sync_copy(ones_hbm, ones_v)
    pltpu.sync_copy(zeros_hbm, row_v)
    pltpu.sync_copy(row_v, acc_d.at[pl.ds(s * RPS, RPS)])
    plsc.subcore_barrier()

    def body(j, carry):
        pltpu.sync_copy(ones_v, acc_d.at[dst_v.at[j]], add=True)
        return carry

    lax.fori_loop(0, CHD, body, 0)
    plsc.subcore_barrier()
    pltpu.sync_copy(acc_d.at[pl.ds(s * RPS, RPS)], row_v)
    pltpu.sync_copy(row_v, out_hbm.at[pl.ds(c * NPAD + s * RPS, RPS)])


@functools.cache
def _deg_kernel():
    mesh = plsc.VectorSubcoreMesh(
        core_axis_name="c", subcore_axis_name="s",
        num_cores=NC, num_subcores=NS)
    return pl.kernel(
        _deg_body,
        out_type=jax.ShapeDtypeStruct((NC * NPAD,), jnp.float32),
        mesh=mesh,
        scratch_types=[
            pltpu.VMEM((CHD, K), jnp.int32),
            pltpu.VMEM((K,), jnp.float32),
            pltpu.VMEM((RPS,), jnp.float32),
            pltpu.VMEM_SHARED((NPAD,), jnp.float32),
        ],
    )


def _agg_body(g_hbm, src_hbm, dst_hbm, zblk_hbm, out_hbm,
              src_v, db, rb, acc, sg0, sg1, sd0, sd1):
    c = lax.axis_index("c")
    s = lax.axis_index("s")
    pltpu.sync_copy(zblk_hbm, rb.at[0])
    for off, sz in _ZCHUNKS:
        pltpu.sync_copy(rb.at[0, pl.ds(0, sz)],
                        acc.at[pl.ds(s * RPS + off, sz)])
    plsc.subcore_barrier()

    sg = (sg0, sg1)
    sd = (sd0, sd1)

    def run_group(base, n):
        pltpu.sync_copy(src_hbm.at[pl.ds(base, n)], src_v.at[pl.ds(0, n)])

        def issue(k, b):
            pltpu.async_copy(g_hbm.at[src_v.at[k]], rb.at[b], sg[b])
            pltpu.async_copy(dst_hbm.at[base + k], db.at[b], sd[b])

        issue(0, 0)

        def body(i, carry):
            for b in range(2):
                k = 2 * i + b

                @pl.when(k + 1 < n)
                def _():
                    issue(k + 1, 1 - b)

                pltpu.make_async_copy(g_hbm.at[src_v.at[k]], rb.at[b],
                                      sg[b]).wait()
                pltpu.make_async_copy(dst_hbm.at[base + k], db.at[b],
                                      sd[b]).wait()
                pltpu.sync_copy(rb.at[b], acc.at[db.at[b]], add=True)
            return carry

        lax.fori_loop(0, n // 2, body, 0)

    @pl.when(c == 0)
    def _():
        for off in range(0, CH0, CHG):
            run_group(s * (CH0 + CH1) + off, CHG)

    @pl.when(c == 1)
    def _():
        run_group(s * (CH0 + CH1) + CH0, CH1)

    plsc.subcore_barrier()
    for off, sz in _ZCHUNKS:
        pltpu.sync_copy(acc.at[pl.ds(s * RPS + off, sz)],
                        rb.at[0, pl.ds(0, sz)])
        pltpu.sync_copy(rb.at[0, pl.ds(0, sz)],
                        out_hbm.at[c, pl.ds(s * RPS + off, sz)])


@functools.cache
def _agg_kernel():
    mesh = plsc.VectorSubcoreMesh(
        core_axis_name="c", subcore_axis_name="s",
        num_cores=NC, num_subcores=NS)
    return pl.kernel(
        _agg_body,
        out_type=jax.ShapeDtypeStruct((NC, NPAD, D), jnp.float32),
        mesh=mesh,
        scratch_types=[
            pltpu.VMEM((CHG, K), jnp.int32),
            pltpu.VMEM((2, K), jnp.int32),
            pltpu.VMEM((2, K, D), jnp.float32),
            pltpu.VMEM_SHARED((NPAD, D), jnp.float32),
        ] + [pltpu.SemaphoreType.DMA] * 4,
    )


_GRID = 8
_R = NPAD // _GRID

_DOT = functools.partial(
    lax.dot_general,
    dimension_numbers=(((1,), (0,)), ((), ())),
    precision=lax.Precision.HIGHEST,
    preferred_element_type=jnp.float32,
)


def _tc1_body(z_ref, d0_ref, d1_ref, g1_ref, dinv_ref):
    deg = d0_ref[...] + d1_ref[...] + 1.0
    dinv = lax.rsqrt(deg)
    dinv_ref[...] = dinv
    g1_ref[...] = dinv * z_ref[...]


def _tc2_body(s_ref, g1_ref, dinv_ref, w1_ref, b1_ref, w2_ref, g2_ref):
    dinv = dinv_ref[...]
    m = dinv * (s_ref[0] + s_ref[1] + g1_ref[...])
    h = jnp.maximum(_DOT(m, w1_ref[...]) + b1_ref[...], 0.0)
    g2_ref[...] = dinv * _DOT(h, w2_ref[...])


def _tc3_body(s_ref, g2_ref, dinv_ref, b2_ref, w3_ref, g3_ref):
    dinv = dinv_ref[...]
    h = jnp.maximum(dinv * (s_ref[0] + s_ref[1] + g2_ref[...]) + b2_ref[...],
                    0.0)
    g3_ref[...] = dinv * _DOT(h, w3_ref[...])


def _tc4_body(s_ref, g3_ref, dinv_ref, b3_ref, out_ref):
    out_ref[...] = (dinv_ref[...] * (s_ref[0] + s_ref[1] + g3_ref[...])
                    + b3_ref[...])


def _rows(bs):
    return pl.BlockSpec(bs, lambda i: (i,) + (0,) * (len(bs) - 1))


def _full(shape):
    return pl.BlockSpec(shape, lambda i: (0,) * len(shape))


def _srows():
    return pl.BlockSpec((NC, _R, D), lambda i: (0, i, 0))


_tc1 = pl.pallas_call(
    _tc1_body,
    grid=(_GRID,),
    in_specs=[_rows((_R, D)), _rows((_R, 1)), _rows((_R, 1))],
    out_specs=[_rows((_R, D)), _rows((_R, 1))],
    out_shape=[jax.ShapeDtypeStruct((NPAD, D), jnp.float32),
               jax.ShapeDtypeStruct((NPAD, 1), jnp.float32)],
)

_tc2 = pl.pallas_call(
    _tc2_body,
    grid=(_GRID,),
    in_specs=[_srows(), _rows((_R, D)), _rows((_R, 1)),
              _full((D, H1)), _full((1, H1)), _full((H1, D))],
    out_specs=_rows((_R, D)),
    out_shape=jax.ShapeDtypeStruct((NPAD, D), jnp.float32),
)

_tc3 = pl.pallas_call(
    _tc3_body,
    grid=(_GRID,),
    in_specs=[_srows(), _rows((_R, D)), _rows((_R, 1)),
              _full((1, D)), _full((D, D))],
    out_specs=_rows((_R, D)),
    out_shape=jax.ShapeDtypeStruct((NPAD, D), jnp.float32),
)

_tc4 = pl.pallas_call(
    _tc4_body,
    grid=(_GRID,),
    in_specs=[_srows(), _rows((_R, D)), _rows((_R, 1)), _full((1, D))],
    out_specs=_rows((_R, D)),
    out_shape=jax.ShapeDtypeStruct((NPAD, D), jnp.float32),
)


def kernel(z, edge_index, W1, b1, W2, b2, W3, b3):
    z_pad = jnp.pad(z, ((0, NPAD - N), (0, 0)))
    pad_idx = jnp.full((EPAD - E,), N, dtype=jnp.int32)
    srcp = jnp.concatenate([edge_index[0], pad_idx]).reshape(TOT_CH, K)
    dstp = jnp.concatenate([edge_index[1], pad_idx]).reshape(TOT_CH, K)
    ones_k = jnp.ones((K,), jnp.float32)
    zeros_n = jnp.zeros((RPS,), jnp.float32)
    zblk = jnp.zeros((K, D), jnp.float32)

    degp = _deg_kernel()(dstp, ones_k, zeros_n)
    d0 = degp[:NPAD][:, None]
    d1 = degp[NPAD:][:, None]

    agg = _agg_kernel()
    g1, dinv = _tc1(z_pad, d0, d1)
    s1 = agg(g1, srcp, dstp, zblk)
    g2 = _tc2(s1, g1, dinv, W1, b1[None, :], W2)
    s2 = agg(g2, srcp, dstp, zblk)
    g3 = _tc3(s2, g2, dinv, b2[None, :], W3)
    s3 = agg(g3, srcp, dstp, zblk)
    out = _tc4(s3, g3, dinv, b3[None, :])
    return out[:N]


# speedup vs baseline: 1.3351x; 1.0931x over previous
import functools

import jax
import jax.numpy as jnp
from jax import lax
from jax.experimental import pallas as pl
from jax.experimental.pallas import tpu as pltpu
import jax.experimental.pallas.tpu_sc as plsc

N = 10000
D = 128
H1 = 256
E = 320000

NC = 2
NS = 16
NW = NC * NS
K = 128
CH0 = 120
CH1 = 40
CHG = 40
TOT_CH = NS * (CH0 + CH1)
EPAD = TOT_CH * K
NPAD = 10112
RPS = NPAD // NS

CHD = TOT_CH // NW

_ZCHUNKS = [(0, 128), (128, 128), (256, 128), (384, 128), (512, 120)]


def _deg_body(dst_hbm, ones_hbm, zeros_hbm, out_hbm, dst_v, ones_v, row_v,
              acc_d):
    c = lax.axis_index("c")
    s = lax.axis_index("s")
    wid = s * NC + c
    pltpu.sync_copy(dst_hbm.at[pl.ds(wid * CHD, CHD)], dst_v)
    pltpu.sync_copy(ones_hbm, ones_v)
    pltpu.sync_copy(zeros_hbm, row_v)
    pltpu.sync_copy(row_v, acc_d.at[pl.ds(s * RPS, RPS)])
    plsc.subcore_barrier()

    def body(j, carry):
        pltpu.sync_copy(ones_v, acc_d.at[dst_v.at[j]], add=True)
        return carry

    lax.fori_loop(0, CHD, body, 0)
    plsc.subcore_barrier()
    pltpu.sync_copy(acc_d.at[pl.ds(s * RPS, RPS)], row_v)
    pltpu.sync_copy(row_v, out_hbm.at[pl.ds(c * NPAD + s * RPS, RPS)])


@functools.cache
def _deg_kernel():
    mesh = plsc.VectorSubcoreMesh(
        core_axis_name="c", subcore_axis_name="s",
        num_cores=NC, num_subcores=NS)
    return pl.kernel(
        _deg_body,
        out_type=jax.ShapeDtypeStruct((NC * NPAD,), jnp.float32),
        mesh=mesh,
        scratch_types=[
            pltpu.VMEM((CHD, K), jnp.int32),
            pltpu.VMEM((K,), jnp.float32),
            pltpu.VMEM((RPS,), jnp.float32),
            pltpu.VMEM_SHARED((NPAD,), jnp.float32),
        ],
    )


def _agg_body(g_hbm, g2_hbm, src_hbm, dst_hbm, zblk_hbm, out_hbm,
              src_v, db, rb, acc, sg0, sg1, sd0, sd1):
    c = lax.axis_index("c")
    s = lax.axis_index("s")
    pltpu.sync_copy(zblk_hbm, rb.at[0])
    for off, sz in _ZCHUNKS:
        pltpu.sync_copy(rb.at[0, pl.ds(0, sz)],
                        acc.at[pl.ds(s * RPS + off, sz)])
    plsc.subcore_barrier()

    sg = (sg0, sg1)
    sd = (sd0, sd1)

    def run_group(base, n, gref):
        pltpu.sync_copy(src_hbm.at[pl.ds(base, n)], src_v.at[pl.ds(0, n)])

        def issue(k, b):
            pltpu.async_copy(gref.at[src_v.at[k]], rb.at[b], sg[b])
            pltpu.async_copy(dst_hbm.at[base + k], db.at[b], sd[b])

        issue(0, 0)

        def body(i, carry):
            for b in range(2):
                k = 2 * i + b

                @pl.when(k + 1 < n)
                def _():
                    issue(k + 1, 1 - b)

                pltpu.make_async_copy(gref.at[src_v.at[k]], rb.at[b],
                                      sg[b]).wait()
                pltpu.make_async_copy(dst_hbm.at[base + k], db.at[b],
                                      sd[b]).wait()
                pltpu.sync_copy(rb.at[b], acc.at[db.at[b]], add=True)
            return carry

        lax.fori_loop(0, n // 2, body, 0)

    @pl.when(c == 0)
    def _():
        for off in range(0, CH0, CHG):
            run_group(s * (CH0 + CH1) + off, CHG, g_hbm)

    @pl.when(c == 1)
    def _():
        run_group(s * (CH0 + CH1) + CH0, CH1, g2_hbm)

    plsc.subcore_barrier()
    for off, sz in _ZCHUNKS:
        pltpu.sync_copy(acc.at[pl.ds(s * RPS + off, sz)],
                        rb.at[0, pl.ds(0, sz)])
        pltpu.sync_copy(rb.at[0, pl.ds(0, sz)],
                        out_hbm.at[c, pl.ds(s * RPS + off, sz)])


@functools.cache
def _agg_kernel():
    mesh = plsc.VectorSubcoreMesh(
        core_axis_name="c", subcore_axis_name="s",
        num_cores=NC, num_subcores=NS)
    return pl.kernel(
        _agg_body,
        out_type=jax.ShapeDtypeStruct((NC, NPAD, D), jnp.float32),
        mesh=mesh,
        scratch_types=[
            pltpu.VMEM((CHG, K), jnp.int32),
            pltpu.VMEM((2, K), jnp.int32),
            pltpu.VMEM((2, K, D), jnp.float32),
            pltpu.VMEM_SHARED((NPAD, D), jnp.float32),
        ] + [pltpu.SemaphoreType.DMA] * 4,
    )


_GRID = 8
_R = NPAD // _GRID

_DOT = functools.partial(
    lax.dot_general,
    dimension_numbers=(((1,), (0,)), ((), ())),
    precision=lax.Precision.HIGHEST,
    preferred_element_type=jnp.float32,
)


def _tc1_body(z_ref, d0_ref, d1_ref, g1_ref, g1c_ref, dinv_ref):
    deg = d0_ref[...] + d1_ref[...] + 1.0
    dinv = lax.rsqrt(deg)
    dinv_ref[...] = dinv
    g1 = dinv * z_ref[...]
    g1_ref[...] = g1
    g1c_ref[...] = g1


def _tc2_body(s_ref, g1_ref, dinv_ref, w1_ref, b1_ref, w2_ref,
              g2_ref, g2c_ref):
    dinv = dinv_ref[...]
    m = dinv * (s_ref[0] + s_ref[1] + g1_ref[...])
    h = jnp.maximum(_DOT(m, w1_ref[...]) + b1_ref[...], 0.0)
    g2 = dinv * _DOT(h, w2_ref[...])
    g2_ref[...] = g2
    g2c_ref[...] = g2


def _tc3_body(s_ref, g2_ref, dinv_ref, b2_ref, w3_ref, g3_ref, g3c_ref):
    dinv = dinv_ref[...]
    h = jnp.maximum(dinv * (s_ref[0] + s_ref[1] + g2_ref[...]) + b2_ref[...],
                    0.0)
    g3 = dinv * _DOT(h, w3_ref[...])
    g3_ref[...] = g3
    g3c_ref[...] = g3


def _tc4_body(s_ref, g3_ref, dinv_ref, b3_ref, out_ref):
    out_ref[...] = (dinv_ref[...] * (s_ref[0] + s_ref[1] + g3_ref[...])
                    + b3_ref[...])


def _rows(bs):
    return pl.BlockSpec(bs, lambda i: (i,) + (0,) * (len(bs) - 1))


def _full(shape):
    return pl.BlockSpec(shape, lambda i: (0,) * len(shape))


def _srows():
    return pl.BlockSpec((NC, _R, D), lambda i: (0, i, 0))


_tc1 = pl.pallas_call(
    _tc1_body,
    grid=(_GRID,),
    in_specs=[_rows((_R, D)), _rows((_R, 1)), _rows((_R, 1))],
    out_specs=[_rows((_R, D)), _rows((_R, D)), _rows((_R, 1))],
    out_shape=[jax.ShapeDtypeStruct((NPAD, D), jnp.float32),
               jax.ShapeDtypeStruct((NPAD, D), jnp.float32),
               jax.ShapeDtypeStruct((NPAD, 1), jnp.float32)],
)

_tc2 = pl.pallas_call(
    _tc2_body,
    grid=(_GRID,),
    in_specs=[_srows(), _rows((_R, D)), _rows((_R, 1)),
              _full((D, H1)), _full((1, H1)), _full((H1, D))],
    out_specs=[_rows((_R, D)), _rows((_R, D))],
    out_shape=[jax.ShapeDtypeStruct((NPAD, D), jnp.float32),
               jax.ShapeDtypeStruct((NPAD, D), jnp.float32)],
)

_tc3 = pl.pallas_call(
    _tc3_body,
    grid=(_GRID,),
    in_specs=[_srows(), _rows((_R, D)), _rows((_R, 1)),
              _full((1, D)), _full((D, D))],
    out_specs=[_rows((_R, D)), _rows((_R, D))],
    out_shape=[jax.ShapeDtypeStruct((NPAD, D), jnp.float32),
               jax.ShapeDtypeStruct((NPAD, D), jnp.float32)],
)

_tc4 = pl.pallas_call(
    _tc4_body,
    grid=(_GRID,),
    in_specs=[_srows(), _rows((_R, D)), _rows((_R, 1)), _full((1, D))],
    out_specs=_rows((_R, D)),
    out_shape=jax.ShapeDtypeStruct((NPAD, D), jnp.float32),
)


def kernel(z, edge_index, W1, b1, W2, b2, W3, b3):
    z_pad = jnp.pad(z, ((0, NPAD - N), (0, 0)))
    pad_idx = jnp.full((EPAD - E,), N, dtype=jnp.int32)
    srcp = jnp.concatenate([edge_index[0], pad_idx]).reshape(TOT_CH, K)
    dstp = jnp.concatenate([edge_index[1], pad_idx]).reshape(TOT_CH, K)
    ones_k = jnp.ones((K,), jnp.float32)
    zeros_n = jnp.zeros((RPS,), jnp.float32)
    zblk = jnp.zeros((K, D), jnp.float32)

    degp = _deg_kernel()(dstp, ones_k, zeros_n)
    d0 = degp[:NPAD][:, None]
    d1 = degp[NPAD:][:, None]

    agg = _agg_kernel()
    g1, g1c, dinv = _tc1(z_pad, d0, d1)
    s1 = agg(g1, g1c, srcp, dstp, zblk)
    g2, g2c = _tc2(s1, g1, dinv, W1, b1[None, :], W2)
    s2 = agg(g2, g2c, srcp, dstp, zblk)
    g3, g3c = _tc3(s2, g2, dinv, b2[None, :], W3)
    s3 = agg(g3, g3c, srcp, dstp, zblk)
    out = _tc4(s3, g3, dinv, b3[None, :])
    return out[:N]
